# Initial kernel scaffold; baseline (speedup 1.0000x reference)
import jax
import jax.numpy as jnp
from jax.experimental import pallas as pl


def kernel(positions, max_neighbours):
    raise NotImplementedError("write your pallas kernel here")


# speedup vs baseline: 26.7112x; 26.7112x over previous
import functools

import jax
import jax.numpy as jnp
from jax import lax
from jax.experimental import pallas as pl
from jax.experimental.pallas import tpu as pltpu
from jax.experimental.pallas import tpu_sc as plsc

N = 1024
MAXN = 80
L = 16
NC, NS = 2, 16
NW = NC * NS
ROWS = N // NW
NCHUNK = N // L
BUF = 128
BIG = 0x3FFFFFFF
SIDE = 25.0
HALF = 12.5
CUT2 = 36.0


def _merge(a, b):
    s = list(a) + [lax.rev(v, (0,)) for v in reversed(list(b))]
    n = len(s)
    stride = n // 2
    while stride >= 1:
        for base in range(0, n, 2 * stride):
            for i in range(base, base + stride):
                lo = jnp.minimum(s[i], s[i + stride])
                hi = jnp.maximum(s[i], s[i + stride])
                s[i], s[i + stride] = lo, hi
        stride //= 2
    return [jnp.sort(v) for v in s]


def _sort_vregs(vs):
    runs = [[jnp.sort(v)] for v in vs]
    while len(runs) > 1:
        runs = [_merge(runs[i], runs[i + 1]) for i in range(0, len(runs), 2)]
    return runs[0]


_mesh = plsc.VectorSubcoreMesh(core_axis_name="c", subcore_axis_name="s")


@functools.partial(
    pl.kernel,
    out_type=(
        jax.ShapeDtypeStruct((N * MAXN,), jnp.int32),
        jax.ShapeDtypeStruct((N * 3 * MAXN,), jnp.int32),
        jax.ShapeDtypeStruct((NW, L), jnp.int32),
    ),
    mesh=_mesh,
    compiler_params=pltpu.CompilerParams(needs_layout_passes=False),
    scratch_types=[
        pltpu.VMEM((N,), jnp.float32),
        pltpu.VMEM((N,), jnp.float32),
        pltpu.VMEM((N,), jnp.float32),
        pltpu.VMEM((BUF,), jnp.int32),
        pltpu.VMEM((ROWS * MAXN,), jnp.int32),
        pltpu.VMEM((ROWS * 3 * MAXN,), jnp.int32),
        pltpu.VMEM((L,), jnp.int32),
    ],
)
def _nbr_kernel(pos_hbm, nbr_hbm, cell_hbm, max_hbm,
                x_v, y_v, z_v, keys_v, nbr_v, cell_v, max_v):
    wid = lax.axis_index("s") * NC + lax.axis_index("c")
    pltpu.sync_copy(pos_hbm.at[pl.ds(0, N)], x_v)
    pltpu.sync_copy(pos_hbm.at[pl.ds(N, N)], y_v)
    pltpu.sync_copy(pos_hbm.at[pl.ds(2 * N, N)], z_v)

    lanes = jnp.arange(L, dtype=jnp.int32)

    def row_body(r, maxcnt):
        q = wid * ROWS + r
        qbase = wid * ROWS + (r // L) * L
        lane_vec = jnp.full((L,), r % L, jnp.int32)
        qx = x_v[pl.ds(qbase, L)].at[lane_vec].get(mode="promise_in_bounds")
        qy = y_v[pl.ds(qbase, L)].at[lane_vec].get(mode="promise_in_bounds")
        qz = z_v[pl.ds(qbase, L)].at[lane_vec].get(mode="promise_in_bounds")

        for j in range(BUF // L):
            keys_v[pl.ds(j * L, L)] = jnp.full((L,), BIG, jnp.int32)

        def chunk_body(i, cnt):
            base = i * L
            px = x_v[pl.ds(base, L)]
            py = y_v[pl.ds(base, L)]
            pz = z_v[pl.ds(base, L)]
            dx = px - qx
            dy = py - qy
            dz = pz - qz
            sx = jnp.where(dx > HALF, -1, 0) + jnp.where(dx < -HALF, 1, 0)
            sy = jnp.where(dy > HALF, -1, 0) + jnp.where(dy < -HALF, 1, 0)
            sz = jnp.where(dz > HALF, -1, 0) + jnp.where(dz < -HALF, 1, 0)
            wx = (px + sx.astype(jnp.float32) * SIDE) - qx
            wy = (py + sy.astype(jnp.float32) * SIDE) - qy
            wz = (pz + sz.astype(jnp.float32) * SIDE) - qz
            d2 = (wx * wx + wy * wy) + wz * wz
            pvec = base + lanes
            hit = (d2 <= CUT2) & (pvec != q)
            cell = (sz + 1) * 9 + (sy + 1) * 3 + (sx + 1)
            key = cell * N + pvec
            inc = jnp.where(hit, 1, 0).astype(jnp.int32)
            pre = plsc.cumsum(inc)
            idx = cnt + pre - 1
            plsc.store_scatter(keys_v, [idx], key, mask=hit & (idx < BUF))
            return cnt + jnp.sum(inc)

        cnt = lax.fori_loop(0, NCHUNK, chunk_body, jnp.int32(0), unroll=2)

        vs = [keys_v[pl.ds(j * L, L)] for j in range(BUF // L)]
        svs = _sort_vregs(vs)
        for j in range(MAXN // L):
            k = svs[j]
            pad = k >= jnp.int32(27 * N)
            p = jnp.where(pad, -1, k & (N - 1))
            c = k >> 10
            cx = jnp.where(pad, 1, c % 3 - 1)
            cy = jnp.where(pad, 1, (c // 3) % 3 - 1)
            cz = jnp.where(pad, 1, c // 9 - 1)
            nbr_v[pl.ds(r * MAXN + j * L, L)] = p
            ii = r * (3 * MAXN) + 3 * j * L + lanes * 3
            plsc.store_scatter(cell_v, [ii], cx)
            plsc.store_scatter(cell_v, [ii + 1], cy)
            plsc.store_scatter(cell_v, [ii + 2], cz)
        return jnp.maximum(maxcnt, cnt)

    maxcnt = lax.fori_loop(0, ROWS, row_body, jnp.int32(0))
    max_v[...] = jnp.full((L,), maxcnt, jnp.int32)
    pltpu.sync_copy(nbr_v, nbr_hbm.at[pl.ds(wid * ROWS * MAXN, ROWS * MAXN)])
    pltpu.sync_copy(cell_v, cell_hbm.at[pl.ds(wid * ROWS * 3 * MAXN, ROWS * 3 * MAXN)])
    pltpu.sync_copy(max_v, max_hbm.at[wid])


def kernel(positions, max_neighbours):
    del max_neighbours
    pos_t = positions.T.astype(jnp.float32).reshape(3 * N)
    nbr_flat, cell_flat, maxs = _nbr_kernel(pos_t)
    neighbours = nbr_flat.reshape(N, MAXN)
    cells = cell_flat.reshape(N, MAXN, 3)
    return neighbours, cells, jnp.max(maxs)


# speedup vs baseline: 30.9730x; 1.1595x over previous
import functools

import jax
import jax.numpy as jnp
from jax import lax
from jax.experimental import pallas as pl
from jax.experimental.pallas import tpu as pltpu
from jax.experimental.pallas import tpu_sc as plsc

N = 1024
MAXN = 80
L = 16
NC, NS = 2, 16
NW = NC * NS
ROWS = N // NW
NCHUNK = N // L
BUF = 128
BIG = 0x3FFFFFFF
SIDE = 25.0
HALF = 12.5
CUT2 = 36.0


def _merge(a, b, need=None):
    s = list(a) + [lax.rev(v, (0,)) for v in reversed(list(b))]
    n = len(s)
    stride = n // 2
    while stride >= 1:
        for base in range(0, n, 2 * stride):
            for i in range(base, base + stride):
                lo = jnp.minimum(s[i], s[i + stride])
                hi = jnp.maximum(s[i], s[i + stride])
                s[i], s[i + stride] = lo, hi
        stride //= 2
    return [jnp.sort(v) if (need is None or i < need) else v
            for i, v in enumerate(s)]


def _sort_vregs(vs, need=None):
    runs = [[jnp.sort(v)] for v in vs]
    while len(runs) > 1:
        last = len(runs) == 2
        runs = [_merge(runs[i], runs[i + 1], need=need if last else None)
                for i in range(0, len(runs), 2)]
    return runs[0]


_mesh = plsc.VectorSubcoreMesh(core_axis_name="c", subcore_axis_name="s")


@functools.partial(
    pl.kernel,
    out_type=(
        jax.ShapeDtypeStruct((N * MAXN,), jnp.int32),
        jax.ShapeDtypeStruct((N * 3 * MAXN,), jnp.int32),
        jax.ShapeDtypeStruct((NW, L), jnp.int32),
    ),
    mesh=_mesh,
    compiler_params=pltpu.CompilerParams(needs_layout_passes=False),
    scratch_types=[
        pltpu.VMEM((N,), jnp.float32),
        pltpu.VMEM((N,), jnp.float32),
        pltpu.VMEM((N,), jnp.float32),
        pltpu.VMEM((BUF,), jnp.int32),
        pltpu.VMEM((ROWS * MAXN,), jnp.int32),
        pltpu.VMEM((ROWS * 3 * MAXN,), jnp.int32),
        pltpu.VMEM((L,), jnp.int32),
    ],
)
def _nbr_kernel(pos_hbm, nbr_hbm, cell_hbm, max_hbm,
                x_v, y_v, z_v, keys_v, nbr_v, cell_v, max_v):
    wid = lax.axis_index("s") * NC + lax.axis_index("c")
    pltpu.sync_copy(pos_hbm.at[pl.ds(0, N)], x_v)
    pltpu.sync_copy(pos_hbm.at[pl.ds(N, N)], y_v)
    pltpu.sync_copy(pos_hbm.at[pl.ds(2 * N, N)], z_v)

    lanes = jnp.arange(L, dtype=jnp.int32)

    def row_body(r, maxcnt):
        q = wid * ROWS + r
        qbase = wid * ROWS + (r // L) * L
        lane_vec = jnp.full((L,), r % L, jnp.int32)
        qx = x_v[pl.ds(qbase, L)].at[lane_vec].get(mode="promise_in_bounds")
        qy = y_v[pl.ds(qbase, L)].at[lane_vec].get(mode="promise_in_bounds")
        qz = z_v[pl.ds(qbase, L)].at[lane_vec].get(mode="promise_in_bounds")

        for j in range(BUF // L):
            keys_v[pl.ds(j * L, L)] = jnp.full((L,), BIG, jnp.int32)

        def chunk_body(i, cnt):
            base = i * L
            px = x_v[pl.ds(base, L)]
            py = y_v[pl.ds(base, L)]
            pz = z_v[pl.ds(base, L)]
            dx = px - qx
            dy = py - qy
            dz = pz - qz
            sx = jnp.where(dx > HALF, -1, 0) + jnp.where(dx < -HALF, 1, 0)
            sy = jnp.where(dy > HALF, -1, 0) + jnp.where(dy < -HALF, 1, 0)
            sz = jnp.where(dz > HALF, -1, 0) + jnp.where(dz < -HALF, 1, 0)
            wx = (px + sx.astype(jnp.float32) * SIDE) - qx
            wy = (py + sy.astype(jnp.float32) * SIDE) - qy
            wz = (pz + sz.astype(jnp.float32) * SIDE) - qz
            d2 = (wx * wx + wy * wy) + wz * wz
            pvec = base + lanes
            hit = (d2 <= CUT2) & (pvec != q)
            cell = (sz + 1) * 9 + (sy + 1) * 3 + (sx + 1)
            key = cell * N + pvec
            inc = jnp.where(hit, 1, 0).astype(jnp.int32)
            pre = plsc.cumsum(inc)
            idx = cnt + pre - 1
            plsc.store_scatter(keys_v, [idx], key, mask=hit & (idx < BUF))
            return cnt + plsc.all_reduce_population_count(hit)

        cnt = lax.fori_loop(0, NCHUNK, chunk_body, jnp.zeros((L,), jnp.int32),
                            unroll=4)

        vs = [keys_v[pl.ds(j * L, L)] for j in range(BUF // L)]
        svs = _sort_vregs(vs, need=MAXN // L)
        for j in range(MAXN // L):
            k = svs[j]
            pad = k >= jnp.int32(27 * N)
            p = jnp.where(pad, -1, k & (N - 1))
            c = k >> 10
            cx = jnp.where(pad, 1, c % 3 - 1)
            cy = jnp.where(pad, 1, (c // 3) % 3 - 1)
            cz = jnp.where(pad, 1, c // 9 - 1)
            nbr_v[pl.ds(r * MAXN + j * L, L)] = p
            ii = r * (3 * MAXN) + 3 * j * L + lanes * 3
            plsc.store_scatter(cell_v, [ii], cx)
            plsc.store_scatter(cell_v, [ii + 1], cy)
            plsc.store_scatter(cell_v, [ii + 2], cz)
        return jnp.maximum(maxcnt, cnt)

    maxcnt = lax.fori_loop(0, ROWS, row_body, jnp.zeros((L,), jnp.int32))
    max_v[...] = maxcnt
    pltpu.sync_copy(nbr_v, nbr_hbm.at[pl.ds(wid * ROWS * MAXN, ROWS * MAXN)])
    pltpu.sync_copy(cell_v, cell_hbm.at[pl.ds(wid * ROWS * 3 * MAXN, ROWS * 3 * MAXN)])
    pltpu.sync_copy(max_v, max_hbm.at[wid])


def kernel(positions, max_neighbours):
    del max_neighbours
    pos_t = positions.T.astype(jnp.float32).reshape(3 * N)
    nbr_flat, cell_flat, maxs = _nbr_kernel(pos_t)
    neighbours = nbr_flat.reshape(N, MAXN)
    cells = cell_flat.reshape(N, MAXN, 3)
    return neighbours, cells, jnp.max(maxs)


# speedup vs baseline: 53.1090x; 1.7147x over previous
import functools

import jax
import jax.numpy as jnp
from jax import lax
from jax.experimental import pallas as pl
from jax.experimental.pallas import tpu as pltpu
from jax.experimental.pallas import tpu_sc as plsc

N = 1024
MAXN = 80
L = 16
NC, NS = 2, 16
NW = NC * NS
ROWS = N // NW
NCHUNK = N // L
BUF = 128
BIG = 0x3FFFFFFF
SIDE = 25.0
HALF = 12.5
CUT2 = 36.0


def _merge(a, b, need=None):
    s = list(a) + [lax.rev(v, (0,)) for v in reversed(list(b))]
    n = len(s)
    stride = n // 2
    while stride >= 1:
        for base in range(0, n, 2 * stride):
            for i in range(base, base + stride):
                lo = jnp.minimum(s[i], s[i + stride])
                hi = jnp.maximum(s[i], s[i + stride])
                s[i], s[i + stride] = lo, hi
        stride //= 2
    return [jnp.sort(v) if (need is None or i < need) else v
            for i, v in enumerate(s)]


def _sort_vregs(vs, need=None):
    runs = [[jnp.sort(v)] for v in vs]
    while len(runs) > 1:
        last = len(runs) == 2
        runs = [_merge(runs[i], runs[i + 1], need=need if last else None)
                for i in range(0, len(runs), 2)]
    return runs[0]


_mesh = plsc.VectorSubcoreMesh(core_axis_name="c", subcore_axis_name="s")


@functools.partial(
    pl.kernel,
    out_type=(
        jax.ShapeDtypeStruct((N * MAXN,), jnp.int32),
        jax.ShapeDtypeStruct((N * MAXN,), jnp.int32),
        jax.ShapeDtypeStruct((N * MAXN,), jnp.int32),
        jax.ShapeDtypeStruct((N * MAXN,), jnp.int32),
        jax.ShapeDtypeStruct((NW, L), jnp.int32),
    ),
    mesh=_mesh,
    compiler_params=pltpu.CompilerParams(needs_layout_passes=False),
    scratch_types=[
        pltpu.VMEM((N,), jnp.float32),
        pltpu.VMEM((N,), jnp.float32),
        pltpu.VMEM((N,), jnp.float32),
        pltpu.VMEM((N,), jnp.int32),
        pltpu.VMEM((ROWS * MAXN,), jnp.int32),
        pltpu.VMEM((ROWS * MAXN,), jnp.int32),
        pltpu.VMEM((ROWS * MAXN,), jnp.int32),
        pltpu.VMEM((ROWS * MAXN,), jnp.int32),
        pltpu.VMEM((L,), jnp.int32),
    ],
)
def _nbr_kernel(pos_hbm, nbr_hbm, cx_hbm, cy_hbm, cz_hbm, max_hbm,
                x_v, y_v, z_v, keys_v, nbr_v, cx_v, cy_v, cz_v, max_v):
    wid = lax.axis_index("s") * NC + lax.axis_index("c")
    pltpu.sync_copy(pos_hbm.at[pl.ds(0, N)], x_v)
    pltpu.sync_copy(pos_hbm.at[pl.ds(N, N)], y_v)
    pltpu.sync_copy(pos_hbm.at[pl.ds(2 * N, N)], z_v)

    lanes = jnp.arange(L, dtype=jnp.int32)

    def row_body(r, maxcnt):
        q = wid * ROWS + r
        qbase = wid * ROWS + (r // L) * L
        lane_vec = jnp.full((L,), r % L, jnp.int32)
        qchunk = x_v[pl.ds(qbase, L)]
        qx = qchunk.at[lane_vec].get(mode="promise_in_bounds")
        qy = y_v[pl.ds(qbase, L)].at[lane_vec].get(mode="promise_in_bounds")
        qz = z_v[pl.ds(qbase, L)].at[lane_vec].get(mode="promise_in_bounds")
        self_lane = lanes == (r % L)
        x_v[pl.ds(qbase, L)] = jnp.where(self_lane, jnp.float32(1e9), qchunk)

        for j in range(BUF // L):
            keys_v[pl.ds(j * L, L)] = jnp.full((L,), BIG, jnp.int32)

        def chunk_body(i, cnt):
            base = i * L
            px = x_v[pl.ds(base, L)]
            py = y_v[pl.ds(base, L)]
            pz = z_v[pl.ds(base, L)]
            dx = px - qx
            dy = py - qy
            dz = pz - qz
            gx, lx = dx > HALF, dx < -HALF
            gy, ly = dy > HALF, dy < -HALF
            gz, lz = dz > HALF, dz < -HALF
            wx = (px + jnp.where(gx, -SIDE, jnp.where(lx, SIDE, 0.0))) - qx
            wy = (py + jnp.where(gy, -SIDE, jnp.where(ly, SIDE, 0.0))) - qy
            wz = (pz + jnp.where(gz, -SIDE, jnp.where(lz, SIDE, 0.0))) - qz
            d2 = (wx * wx + wy * wy) + wz * wz
            pvec = base + lanes
            hit = d2 <= CUT2
            key = (13 * N + pvec
                   + jnp.where(gx, -N, 0) + jnp.where(lx, N, 0)
                   + jnp.where(gy, -3 * N, 0) + jnp.where(ly, 3 * N, 0)
                   + jnp.where(gz, -9 * N, 0) + jnp.where(lz, 9 * N, 0))
            inc = hit.astype(jnp.int32)
            pre = plsc.cumsum(inc)
            idx = cnt + pre - 1
            plsc.store_scatter(keys_v, [idx], key, mask=hit)
            return cnt + plsc.all_reduce_population_count(hit)

        cnt = lax.fori_loop(0, NCHUNK, chunk_body, jnp.zeros((L,), jnp.int32),
                            unroll=4)
        x_v[pl.ds(qbase, L)] = qchunk

        vs = [keys_v[pl.ds(j * L, L)] for j in range(BUF // L)]
        svs = _sort_vregs(vs, need=MAXN // L)
        for j in range(MAXN // L):
            k = svs[j]
            pad = k >= jnp.int32(27 * N)
            p = jnp.where(pad, -1, k & (N - 1))
            c = k >> 10
            ob = r * MAXN + j * L
            nbr_v[pl.ds(ob, L)] = p
            cx_v[pl.ds(ob, L)] = jnp.where(pad, 1, c % 3 - 1)
            cy_v[pl.ds(ob, L)] = jnp.where(pad, 1, (c // 3) % 3 - 1)
            cz_v[pl.ds(ob, L)] = jnp.where(pad, 1, c // 9 - 1)
        return jnp.maximum(maxcnt, cnt)

    maxcnt = lax.fori_loop(0, ROWS, row_body, jnp.zeros((L,), jnp.int32))
    max_v[...] = maxcnt
    span = pl.ds(wid * ROWS * MAXN, ROWS * MAXN)
    pltpu.sync_copy(nbr_v, nbr_hbm.at[span])
    pltpu.sync_copy(cx_v, cx_hbm.at[span])
    pltpu.sync_copy(cy_v, cy_hbm.at[span])
    pltpu.sync_copy(cz_v, cz_hbm.at[span])
    pltpu.sync_copy(max_v, max_hbm.at[wid])


def kernel(positions, max_neighbours):
    del max_neighbours
    pos_t = positions.T.astype(jnp.float32).reshape(3 * N)
    nbr_flat, cx_flat, cy_flat, cz_flat, maxs = _nbr_kernel(pos_t)
    neighbours = nbr_flat.reshape(N, MAXN)
    cells = jnp.stack(
        [cx_flat.reshape(N, MAXN), cy_flat.reshape(N, MAXN),
         cz_flat.reshape(N, MAXN)], axis=-1)
    return neighbours, cells, jnp.max(maxs)
